# Initial kernel scaffold; baseline (speedup 1.0000x reference)
import jax
import jax.numpy as jnp
from jax.experimental import pallas as pl


def kernel(h, pos, edge_index, a_ij, We1, be1, We2, be2, Wc1, bc1, Wc2, Wn1, bn1, Wn2, bn2):
    raise NotImplementedError("write your pallas kernel here")


# speedup vs baseline: 2.5338x; 2.5338x over previous
import functools

import jax
import jax.numpy as jnp
from jax import lax
from jax.experimental import pallas as pl
from jax.experimental.pallas import tpu as pltpu
from jax.experimental.pallas import tpu_sc as plsc

N = 10000
E = 320000
DIM = 128
EDGE_DIM = 4
NUM_CONVS = 4
AVG_DEG = E // N

NC = 2
NS = 16
NW = NC * NS
L = 16
CHUNK = 128
NCHUNKS = E // CHUNK
PW = 4
NPAD = 10240
RPT = NPAD // NS

BE = 2000
BN = 2000

f32 = jnp.float32
i32 = jnp.int32


def _silu(x):
    return x * jax.nn.sigmoid(x)


@functools.lru_cache(maxsize=None)
def _sc_mesh():
    return plsc.VectorSubcoreMesh(core_axis_name="c", subcore_axis_name="s",
                                  num_cores=NC, num_subcores=NS)


def _gather_body(xs_hbm, xd_hbm, pos_hbm, src_hbm, dst_hbm,
                 gs_out, gd_out, rel_out,
                 pos_v, si, di, bs, bd, brel, gsem, wsem):
    c = lax.axis_index("c")
    s = lax.axis_index("s")
    wid = s * NC + c
    nk = (NCHUNKS - wid + NW - 1) // NW

    pltpu.sync_copy(pos_hbm, pos_v)
    zero = jnp.zeros((L,), f32)
    for j in range(CHUNK * PW // L):
        brel[pl.ds(j * L, L)] = zero

    lanes = lax.iota(i32, L)

    def body(i, carry):
        k = wid + i * NW
        base = k * CHUNK
        pltpu.sync_copy(src_hbm.at[pl.ds(base, CHUNK)], si)
        pltpu.sync_copy(dst_hbm.at[pl.ds(base, CHUNK)], di)
        c1 = pltpu.async_copy(xs_hbm.at[si], bs, gsem)
        c2 = pltpu.async_copy(xd_hbm.at[di], bd, gsem)
        for g in range(CHUNK // L):
            sv = si[pl.ds(g * L, L)] * PW
            dv = di[pl.ds(g * L, L)] * PW
            ab = (g * L + lanes) * PW
            for cc in range(3):
                vs = plsc.load_gather(pos_v, [sv + cc])
                vd = plsc.load_gather(pos_v, [dv + cc])
                plsc.store_scatter(brel, [ab + cc], vs - vd)
        c1.wait()
        c2.wait()
        w1 = pltpu.async_copy(bs, gs_out.at[pl.ds(base, CHUNK)], wsem)
        w2 = pltpu.async_copy(bd, gd_out.at[pl.ds(base, CHUNK)], wsem)
        w3 = pltpu.async_copy(brel, rel_out.at[pl.ds(base * PW, CHUNK * PW)],
                              wsem)
        w1.wait()
        w2.wait()
        w3.wait()
        return carry

    lax.fori_loop(0, nk, body, 0)


@functools.lru_cache(maxsize=None)
def _gather_kernel():
    return pl.kernel(
        _gather_body,
        mesh=_sc_mesh(),
        out_type=[
            jax.ShapeDtypeStruct((E, DIM), f32),
            jax.ShapeDtypeStruct((E, DIM), f32),
            jax.ShapeDtypeStruct((E * PW,), f32),
        ],
        scratch_types=[
            pltpu.VMEM((NPAD * PW,), f32),
            pltpu.VMEM((CHUNK,), i32),
            pltpu.VMEM((CHUNK,), i32),
            pltpu.VMEM((CHUNK, DIM), f32),
            pltpu.VMEM((CHUNK, DIM), f32),
            pltpu.VMEM((CHUNK * PW,), f32),
            pltpu.SemaphoreType.DMA,
            pltpu.SemaphoreType.DMA,
        ],
        compiler_params=pltpu.CompilerParams(needs_layout_passes=False),
    )


def _gather(xs, xd, pos_flat, src, dst):
    return _gather_kernel()(xs, xd, pos_flat, src, dst)


S_IN = 2 * DIM + 1 + EDGE_DIM


def _edge_body(gs_ref, gd_ref, r_ref, a_ref,
               w1_ref, b1_ref, w2_ref, b2_ref,
               wc1_ref, bc1_ref, wc2_ref,
               m_ref, t_ref):
    r = r_ref[...]
    cmask = (lax.broadcasted_iota(i32, (1, PW), 1) < 3).astype(f32)
    d2 = jnp.sum(r * r * cmask, axis=1, keepdims=True)
    cat = jnp.concatenate([gs_ref[...], gd_ref[...], d2, a_ref[...]], axis=1)
    m1 = _silu(jnp.dot(cat, w1_ref[...], preferred_element_type=f32)
               + b1_ref[...])
    m = _silu(jnp.dot(m1, w2_ref[...], preferred_element_type=f32)
              + b2_ref[...])
    g = _silu(jnp.dot(m, wc1_ref[...], preferred_element_type=f32)
              + bc1_ref[...])
    wgt = jnp.dot(g, wc2_ref[...], preferred_element_type=f32)[:, :1]
    m_ref[...] = m
    t_ref[...] = r * wgt


def _edge(gs, gd, r, a_ij, w1, b1, w2, b2, wc1, bc1, wc2):
    full = lambda shp: pl.BlockSpec(shp, lambda i: tuple(0 for _ in shp))
    return pl.pallas_call(
        _edge_body,
        grid=(E // BE,),
        in_specs=[
            pl.BlockSpec((BE, DIM), lambda i: (i, 0)),
            pl.BlockSpec((BE, DIM), lambda i: (i, 0)),
            pl.BlockSpec((BE, PW), lambda i: (i, 0)),
            pl.BlockSpec((BE, EDGE_DIM), lambda i: (i, 0)),
            full((S_IN, DIM)),
            full((1, DIM)),
            full((DIM, DIM)),
            full((1, DIM)),
            full((DIM, DIM)),
            full((1, DIM)),
            full((DIM, 8)),
        ],
        out_specs=[
            pl.BlockSpec((BE, DIM), lambda i: (i, 0)),
            pl.BlockSpec((BE, PW), lambda i: (i, 0)),
        ],
        out_shape=[
            jax.ShapeDtypeStruct((E, DIM), f32),
            jax.ShapeDtypeStruct((E, PW), f32),
        ],
    )(gs, gd, r, a_ij, w1, b1, w2, b2, wc1, bc1, wc2)

def _scatter_m_body(m_hbm, dst_hbm, zm_hbm, pm_out, acc_m, di, bm, sem):
    c = lax.axis_index("c")
    s = lax.axis_index("s")
    wid = s * NC + c
    row0 = s * RPT
    pltpu.sync_copy(zm_hbm, acc_m.at[pl.ds(row0, RPT)])
    plsc.subcore_barrier()

    nk = (NCHUNKS - wid + NW - 1) // NW

    def body(i, carry):
        k = wid + i * NW
        base = k * CHUNK
        pltpu.sync_copy(dst_hbm.at[pl.ds(base, CHUNK)], di)
        pltpu.async_copy(m_hbm.at[pl.ds(base, CHUNK)], bm, sem).wait()
        pltpu.sync_copy(bm, acc_m.at[di], add=True)
        return carry

    lax.fori_loop(0, nk, body, 0)
    plsc.subcore_barrier()
    pltpu.sync_copy(acc_m.at[pl.ds(row0, RPT)],
                    pm_out.at[c].at[pl.ds(row0, RPT)])


@functools.lru_cache(maxsize=None)
def _scatter_m_kernel():
    return pl.kernel(
        _scatter_m_body,
        mesh=_sc_mesh(),
        out_type=jax.ShapeDtypeStruct((NC, NPAD, DIM), f32),
        scratch_types=[
            pltpu.VMEM_SHARED((NPAD, DIM), f32),
            pltpu.VMEM((CHUNK,), i32),
            pltpu.VMEM((CHUNK, DIM), f32),
            pltpu.SemaphoreType.DMA,
        ],
        compiler_params=pltpu.CompilerParams(needs_layout_passes=False),
    )


def _scatter_t_body(t_hbm, dst_hbm, zm_hbm, pt_out, acc_t, di, bt, bex, sem):
    c = lax.axis_index("c")
    s = lax.axis_index("s")
    wid = s * NC + c
    row0 = s * RPT
    pltpu.sync_copy(zm_hbm, acc_t.at[pl.ds(row0, RPT)])

    zero = jnp.zeros((L,), f32)

    def zrow(r, carry):
        for gg in range(DIM // L):
            bex[r, pl.ds(gg * L, L)] = zero
        return carry

    lax.fori_loop(0, CHUNK, zrow, 0)
    plsc.subcore_barrier()

    nk = (NCHUNKS - wid + NW - 1) // NW
    lanes = lax.iota(i32, L)

    def body(i, carry):
        k = wid + i * NW
        base = k * CHUNK
        pltpu.sync_copy(dst_hbm.at[pl.ds(base, CHUNK)], di)
        pltpu.async_copy(t_hbm.at[pl.ds(base * PW, CHUNK * PW)], bt, sem).wait()
        for g in range(CHUNK // L):
            rows = g * L + lanes
            ab = rows * PW
            for cc in range(3):
                vals = plsc.load_gather(bt, [ab + cc])
                plsc.store_scatter(bex, [rows, jnp.full((L,), cc, i32)], vals)
        pltpu.sync_copy(bex, acc_t.at[di], add=True)
        return carry

    lax.fori_loop(0, nk, body, 0)
    plsc.subcore_barrier()
    pltpu.sync_copy(acc_t.at[pl.ds(row0, RPT)],
                    pt_out.at[c].at[pl.ds(row0, RPT)])


@functools.lru_cache(maxsize=None)
def _scatter_t_kernel():
    return pl.kernel(
        _scatter_t_body,
        mesh=_sc_mesh(),
        out_type=jax.ShapeDtypeStruct((NC, NPAD, DIM), f32),
        scratch_types=[
            pltpu.VMEM_SHARED((NPAD, DIM), f32),
            pltpu.VMEM((CHUNK,), i32),
            pltpu.VMEM((CHUNK * PW,), f32),
            pltpu.VMEM((CHUNK, DIM), f32),
            pltpu.SemaphoreType.DMA,
        ],
        compiler_params=pltpu.CompilerParams(needs_layout_passes=False),
    )


def _scatter(m, t_flat, dst, zm):
    pm = _scatter_m_kernel()(m, dst, zm)
    pt = _scatter_t_kernel()(t_flat, dst, zm)
    return pm, pt


def _node_body(h_ref, pm0_ref, pm1_ref,
               wn1_ref, bn1_ref, wn2_ref, bn2_ref,
               ho_ref):
    h = h_ref[...]
    agg = pm0_ref[...] + pm1_ref[...]
    cat = jnp.concatenate([h, agg], axis=1)
    u = _silu(jnp.dot(cat, wn1_ref[...], preferred_element_type=f32)
              + bn1_ref[...])
    upd = jnp.dot(u, wn2_ref[...], preferred_element_type=f32) + bn2_ref[...]
    ho_ref[...] = h + upd


def _node(h, pm0, pm1, wn1, bn1, wn2, bn2):
    full = lambda shp: pl.BlockSpec(shp, lambda i: tuple(0 for _ in shp))
    return pl.pallas_call(
        _node_body,
        grid=(N // BN,),
        in_specs=[
            pl.BlockSpec((BN, DIM), lambda i: (i, 0)),
            pl.BlockSpec((BN, DIM), lambda i: (i, 0)),
            pl.BlockSpec((BN, DIM), lambda i: (i, 0)),
            full((2 * DIM, DIM)),
            full((1, DIM)),
            full((DIM, DIM)),
            full((1, DIM)),
        ],
        out_specs=pl.BlockSpec((BN, DIM), lambda i: (i, 0)),
        out_shape=jax.ShapeDtypeStruct((N, DIM), f32),
    )(h, pm0, pm1, wn1, bn1, wn2, bn2)


BNP = 2048


def _pos_body(p_ref, pt_ref, po_ref):
    tsum = pt_ref[0] + pt_ref[1]
    po_ref[...] = p_ref[...] + tsum[:, :PW] * (1.0 / AVG_DEG)


def _pos_update(posf, pt):
    return pl.pallas_call(
        _pos_body,
        grid=(NPAD // BNP,),
        in_specs=[
            pl.BlockSpec((BNP, PW), lambda i: (i, 0)),
            pl.BlockSpec((NC, BNP, DIM), lambda i: (0, i, 0)),
        ],
        out_specs=pl.BlockSpec((BNP, PW), lambda i: (i, 0)),
        out_shape=jax.ShapeDtypeStruct((NPAD, PW), f32),
    )(posf.reshape(NPAD, PW), pt).reshape(-1)


def kernel(h, pos, edge_index, a_ij, We1, be1, We2, be2, Wc1, bc1, Wc2,
           Wn1, bn1, Wn2, bn2):
    src = edge_index[0]
    dst = edge_index[1]
    posf = jnp.pad(pos, ((0, NPAD - N), (0, PW - 3))).reshape(-1)
    zm = jnp.zeros((RPT, DIM), f32)

    for i in range(NUM_CONVS):
        gs, gd, rel_flat = _gather(h, h, posf, src, dst)
        m, t = _edge(gs, gd, rel_flat.reshape(E, PW), a_ij,
                     We1[i], be1[i][None], We2[i], be2[i][None],
                     Wc1[i], bc1[i][None],
                     jnp.pad(Wc2[i], ((0, 0), (0, 7))))
        pm, pt = _scatter(m, t.reshape(-1), dst, zm)
        posf = _pos_update(posf, pt)
        h = _node(h, pm[0], pm[1], Wn1[i], bn1[i][None],
                  Wn2[i], bn2[i][None])
    return h, posf.reshape(NPAD, PW)[:N, :3]


# speedup vs baseline: 2.8073x; 1.1079x over previous
import functools

import jax
import jax.numpy as jnp
from jax import lax
from jax.experimental import pallas as pl
from jax.experimental.pallas import tpu as pltpu
from jax.experimental.pallas import tpu_sc as plsc

N = 10000
E = 320000
DIM = 128
EDGE_DIM = 4
NUM_CONVS = 4
AVG_DEG = E // N

NC = 2
NS = 16
NW = NC * NS
L = 16
CHUNK = 128
NCHUNKS = E // CHUNK
PW = 4
TW = 8
NPAD = 10240
RPT = NPAD // NS

BE = 2000
BN = 2000

f32 = jnp.float32
i32 = jnp.int32


def _silu(x):
    return x * jax.nn.sigmoid(x)


@functools.lru_cache(maxsize=None)
def _sc_mesh():
    return plsc.VectorSubcoreMesh(core_axis_name="c", subcore_axis_name="s",
                                  num_cores=NC, num_subcores=NS)


def _gather_body(xs_hbm, xd_hbm, pos_hbm, src_hbm, dst_hbm,
                 gs_out, gd_out, rel_out,
                 pos_v, si, di, bs, bd, brel, gsem, wsem):
    c = lax.axis_index("c")
    s = lax.axis_index("s")
    wid = s * NC + c
    nk = (NCHUNKS - wid + NW - 1) // NW

    pltpu.sync_copy(pos_hbm, pos_v)
    zero = jnp.zeros((L,), f32)
    for j in range(CHUNK * PW // L):
        brel[pl.ds(j * L, L)] = zero

    lanes = lax.iota(i32, L)

    def body(i, carry):
        k = wid + i * NW
        base = k * CHUNK
        pltpu.sync_copy(src_hbm.at[pl.ds(base, CHUNK)], si)
        pltpu.sync_copy(dst_hbm.at[pl.ds(base, CHUNK)], di)
        c1 = pltpu.async_copy(xs_hbm.at[si], bs, gsem)
        c2 = pltpu.async_copy(xd_hbm.at[di], bd, gsem)
        for g in range(CHUNK // L):
            sv = si[pl.ds(g * L, L)] * PW
            dv = di[pl.ds(g * L, L)] * PW
            ab = (g * L + lanes) * PW
            for cc in range(3):
                vs = plsc.load_gather(pos_v, [sv + cc])
                vd = plsc.load_gather(pos_v, [dv + cc])
                plsc.store_scatter(brel, [ab + cc], vs - vd)
        c1.wait()
        c2.wait()
        w1 = pltpu.async_copy(bs, gs_out.at[pl.ds(base, CHUNK)], wsem)
        w2 = pltpu.async_copy(bd, gd_out.at[pl.ds(base, CHUNK)], wsem)
        w3 = pltpu.async_copy(brel, rel_out.at[pl.ds(base * PW, CHUNK * PW)],
                              wsem)
        w1.wait()
        w2.wait()
        w3.wait()
        return carry

    lax.fori_loop(0, nk, body, 0)


@functools.lru_cache(maxsize=None)
def _gather_kernel():
    return pl.kernel(
        _gather_body,
        mesh=_sc_mesh(),
        out_type=[
            jax.ShapeDtypeStruct((E, DIM), f32),
            jax.ShapeDtypeStruct((E, DIM), f32),
            jax.ShapeDtypeStruct((E * PW,), f32),
        ],
        scratch_types=[
            pltpu.VMEM((NPAD * PW,), f32),
            pltpu.VMEM((CHUNK,), i32),
            pltpu.VMEM((CHUNK,), i32),
            pltpu.VMEM((CHUNK, DIM), f32),
            pltpu.VMEM((CHUNK, DIM), f32),
            pltpu.VMEM((CHUNK * PW,), f32),
            pltpu.SemaphoreType.DMA,
            pltpu.SemaphoreType.DMA,
        ],
        compiler_params=pltpu.CompilerParams(needs_layout_passes=False),
    )


def _gather(xs, xd, pos_flat, src, dst):
    return _gather_kernel()(xs, xd, pos_flat, src, dst)


S_IN = 2 * DIM + 1 + EDGE_DIM


def _edge_body(gs_ref, gd_ref, r_ref, a_ref,
               w1_ref, b1_ref, w2_ref, b2_ref,
               wc1_ref, bc1_ref, wc2_ref,
               m_ref, t_ref):
    r = r_ref[...]
    cmask = (lax.broadcasted_iota(i32, (1, PW), 1) < 3).astype(f32)
    d2 = jnp.sum(r * r * cmask, axis=1, keepdims=True)
    cat = jnp.concatenate([gs_ref[...], gd_ref[...], d2, a_ref[...]], axis=1)
    m1 = _silu(jnp.dot(cat, w1_ref[...], preferred_element_type=f32)
               + b1_ref[...])
    m = _silu(jnp.dot(m1, w2_ref[...], preferred_element_type=f32)
              + b2_ref[...])
    g = _silu(jnp.dot(m, wc1_ref[...], preferred_element_type=f32)
              + bc1_ref[...])
    wgt = jnp.dot(g, wc2_ref[...], preferred_element_type=f32)[:, :1]
    m_ref[...] = m
    t_ref[...] = jnp.concatenate([r * wgt, jnp.zeros_like(r)], axis=1)


def _edge(gs, gd, r, a_ij, w1, b1, w2, b2, wc1, bc1, wc2):
    full = lambda shp: pl.BlockSpec(shp, lambda i: tuple(0 for _ in shp))
    return pl.pallas_call(
        _edge_body,
        grid=(E // BE,),
        in_specs=[
            pl.BlockSpec((BE, DIM), lambda i: (i, 0)),
            pl.BlockSpec((BE, DIM), lambda i: (i, 0)),
            pl.BlockSpec((BE, PW), lambda i: (i, 0)),
            pl.BlockSpec((BE, EDGE_DIM), lambda i: (i, 0)),
            full((S_IN, DIM)),
            full((1, DIM)),
            full((DIM, DIM)),
            full((1, DIM)),
            full((DIM, DIM)),
            full((1, DIM)),
            full((DIM, 8)),
        ],
        out_specs=[
            pl.BlockSpec((BE, DIM), lambda i: (i, 0)),
            pl.BlockSpec((BE, TW), lambda i: (i, 0)),
        ],
        out_shape=[
            jax.ShapeDtypeStruct((E, DIM), f32),
            jax.ShapeDtypeStruct((E, TW), f32),
        ],
    )(gs, gd, r, a_ij, w1, b1, w2, b2, wc1, bc1, wc2)

def _scatter_body(m_hbm, t_hbm, dst_hbm, zm_hbm, zt_hbm,
                  pm_out, pt_out,
                  acc_m, acc_t, di, bm, bt, sem):
    c = lax.axis_index("c")
    s = lax.axis_index("s")
    wid = s * NC + c
    row0 = s * RPT
    pltpu.sync_copy(zm_hbm, acc_m.at[pl.ds(row0, RPT)])
    pltpu.sync_copy(zt_hbm, acc_t.at[pl.ds(row0, RPT)])
    plsc.subcore_barrier()

    nk = (NCHUNKS - wid + NW - 1) // NW

    def body(i, carry):
        k = wid + i * NW
        base = k * CHUNK
        pltpu.sync_copy(dst_hbm.at[pl.ds(base, CHUNK)], di)
        c1 = pltpu.async_copy(m_hbm.at[pl.ds(base, CHUNK)], bm, sem)
        c2 = pltpu.async_copy(t_hbm.at[pl.ds(base, CHUNK)], bt, sem)
        c1.wait()
        c2.wait()
        pltpu.sync_copy(bm, acc_m.at[di], add=True)
        pltpu.sync_copy(bt, acc_t.at[di], add=True)
        return carry

    lax.fori_loop(0, nk, body, 0)
    plsc.subcore_barrier()
    pltpu.sync_copy(acc_m.at[pl.ds(row0, RPT)],
                    pm_out.at[c].at[pl.ds(row0, RPT)])
    pltpu.sync_copy(acc_t.at[pl.ds(row0, RPT)],
                    pt_out.at[c].at[pl.ds(row0, RPT)])


@functools.lru_cache(maxsize=None)
def _scatter_kernel():
    return pl.kernel(
        _scatter_body,
        mesh=_sc_mesh(),
        out_type=[
            jax.ShapeDtypeStruct((NC, NPAD, DIM), f32),
            jax.ShapeDtypeStruct((NC, NPAD, TW), f32),
        ],
        scratch_types=[
            pltpu.VMEM_SHARED((NPAD, DIM), f32),
            pltpu.VMEM_SHARED((NPAD, TW), f32),
            pltpu.VMEM((CHUNK,), i32),
            pltpu.VMEM((CHUNK, DIM), f32),
            pltpu.VMEM((CHUNK, TW), f32),
            pltpu.SemaphoreType.DMA,
        ],
        compiler_params=pltpu.CompilerParams(needs_layout_passes=False,
                                             use_tc_tiling_on_sc=False),
    )


def _scatter(m, t, dst, zm, zt):
    return _scatter_kernel()(m, t, dst, zm, zt)


def _node_body(h_ref, pm0_ref, pm1_ref,
               wn1_ref, bn1_ref, wn2_ref, bn2_ref,
               ho_ref):
    h = h_ref[...]
    agg = pm0_ref[...] + pm1_ref[...]
    cat = jnp.concatenate([h, agg], axis=1)
    u = _silu(jnp.dot(cat, wn1_ref[...], preferred_element_type=f32)
              + bn1_ref[...])
    upd = jnp.dot(u, wn2_ref[...], preferred_element_type=f32) + bn2_ref[...]
    ho_ref[...] = h + upd


def _node(h, pm0, pm1, wn1, bn1, wn2, bn2):
    full = lambda shp: pl.BlockSpec(shp, lambda i: tuple(0 for _ in shp))
    return pl.pallas_call(
        _node_body,
        grid=(N // BN,),
        in_specs=[
            pl.BlockSpec((BN, DIM), lambda i: (i, 0)),
            pl.BlockSpec((BN, DIM), lambda i: (i, 0)),
            pl.BlockSpec((BN, DIM), lambda i: (i, 0)),
            full((2 * DIM, DIM)),
            full((1, DIM)),
            full((DIM, DIM)),
            full((1, DIM)),
        ],
        out_specs=pl.BlockSpec((BN, DIM), lambda i: (i, 0)),
        out_shape=jax.ShapeDtypeStruct((N, DIM), f32),
    )(h, pm0, pm1, wn1, bn1, wn2, bn2)


BNP = 2048


def _pos_body(p_ref, pt_ref, po_ref):
    tsum = pt_ref[0] + pt_ref[1]
    po_ref[...] = p_ref[...] + tsum[:, :PW] * (1.0 / AVG_DEG)


def _pos_update(posf, pt):
    return pl.pallas_call(
        _pos_body,
        grid=(NPAD // BNP,),
        in_specs=[
            pl.BlockSpec((BNP, PW), lambda i: (i, 0)),
            pl.BlockSpec((NC, BNP, TW), lambda i: (0, i, 0)),
        ],
        out_specs=pl.BlockSpec((BNP, PW), lambda i: (i, 0)),
        out_shape=jax.ShapeDtypeStruct((NPAD, PW), f32),
    )(posf.reshape(NPAD, PW), pt).reshape(-1)


def kernel(h, pos, edge_index, a_ij, We1, be1, We2, be2, Wc1, bc1, Wc2,
           Wn1, bn1, Wn2, bn2):
    src = edge_index[0]
    dst = edge_index[1]
    posf = jnp.pad(pos, ((0, NPAD - N), (0, PW - 3))).reshape(-1)
    zm = jnp.zeros((RPT, DIM), f32)
    zt = jnp.zeros((RPT, TW), f32)

    for i in range(NUM_CONVS):
        gs, gd, rel_flat = _gather(h, h, posf, src, dst)
        m, t = _edge(gs, gd, rel_flat.reshape(E, PW), a_ij,
                     We1[i], be1[i][None], We2[i], be2[i][None],
                     Wc1[i], bc1[i][None],
                     jnp.pad(Wc2[i], ((0, 0), (0, 7))))
        pm, pt = _scatter(m, t, dst, zm, zt)
        posf = _pos_update(posf, pt)
        h = _node(h, pm[0], pm[1], Wn1[i], bn1[i][None],
                  Wn2[i], bn2[i][None])
    return h, posf.reshape(NPAD, PW)[:N, :3]


# speedup vs baseline: 2.9298x; 1.0436x over previous
import functools

import jax
import jax.numpy as jnp
from jax import lax
from jax.experimental import pallas as pl
from jax.experimental.pallas import tpu as pltpu
from jax.experimental.pallas import tpu_sc as plsc

N = 10000
E = 320000
DIM = 128
EDGE_DIM = 4
NUM_CONVS = 4
AVG_DEG = E // N

NC = 2
NS = 16
NW = NC * NS
L = 16
CHUNK = 128
NCHUNKS = E // CHUNK
PW = 8
TW = 8
NPAD = 10240
RPT = NPAD // NS

BE = 2000
BN = 2000

f32 = jnp.float32
i32 = jnp.int32


def _silu(x):
    return x * jax.nn.sigmoid(x)


@functools.lru_cache(maxsize=None)
def _sc_mesh():
    return plsc.VectorSubcoreMesh(core_axis_name="c", subcore_axis_name="s",
                                  num_cores=NC, num_subcores=NS)


def _gather_body(xs_hbm, pos_hbm, src_hbm, dst_hbm,
                 gs_out, gd_out, rel_out,
                 pos_v, si, di, bs, bd, brel, gsem, wsem):
    c = lax.axis_index("c")
    s = lax.axis_index("s")
    wid = s * NC + c
    nk = (NCHUNKS - wid + NW - 1) // NW

    pltpu.sync_copy(pos_hbm, pos_v)
    lanes = lax.iota(i32, L)
    zero = jnp.zeros((L,), f32)
    for j in range(CHUNK * PW // L):
        r2 = 2 * j + lax.shift_right_logical(lanes, 3)
        c2 = lanes & 7
        plsc.store_scatter(brel, [r2, c2], zero)

    def body(i, carry):
        k = wid + i * NW
        base = k * CHUNK
        pltpu.sync_copy(src_hbm.at[pl.ds(base, CHUNK)], si)
        pltpu.sync_copy(dst_hbm.at[pl.ds(base, CHUNK)], di)
        c1 = pltpu.async_copy(xs_hbm.at[si], bs, gsem)
        c2 = pltpu.async_copy(xs_hbm.at[di], bd, gsem)
        for g in range(CHUNK // L):
            sv = si[pl.ds(g * L, L)]
            dv = di[pl.ds(g * L, L)]
            rows = g * L + lanes
            for cc in range(3):
                cvec = jnp.full((L,), cc, i32)
                vs = plsc.load_gather(pos_v, [sv, cvec])
                vd = plsc.load_gather(pos_v, [dv, cvec])
                plsc.store_scatter(brel, [rows, cvec], vs - vd)
        c1.wait()
        c2.wait()
        w1 = pltpu.async_copy(bs, gs_out.at[pl.ds(base, CHUNK)], wsem)
        w2 = pltpu.async_copy(bd, gd_out.at[pl.ds(base, CHUNK)], wsem)
        w3 = pltpu.async_copy(brel, rel_out.at[pl.ds(base, CHUNK)], wsem)
        w1.wait()
        w2.wait()
        w3.wait()
        return carry

    lax.fori_loop(0, nk, body, 0)


@functools.lru_cache(maxsize=None)
def _gather_kernel():
    return pl.kernel(
        _gather_body,
        mesh=_sc_mesh(),
        out_type=[
            jax.ShapeDtypeStruct((E, DIM), f32),
            jax.ShapeDtypeStruct((E, DIM), f32),
            jax.ShapeDtypeStruct((E, PW), f32),
        ],
        scratch_types=[
            pltpu.VMEM((NPAD, PW), f32),
            pltpu.VMEM((CHUNK,), i32),
            pltpu.VMEM((CHUNK,), i32),
            pltpu.VMEM((CHUNK, DIM), f32),
            pltpu.VMEM((CHUNK, DIM), f32),
            pltpu.VMEM((CHUNK, PW), f32),
            pltpu.SemaphoreType.DMA,
            pltpu.SemaphoreType.DMA,
        ],
        compiler_params=pltpu.CompilerParams(needs_layout_passes=False,
                                             use_tc_tiling_on_sc=False),
    )


def _gather(h, pos, src, dst):
    return _gather_kernel()(h, pos, src, dst)


S_IN = 2 * DIM + 1 + EDGE_DIM


def _edge_body(gs_ref, gd_ref, r_ref, a_ref,
               w1_ref, b1_ref, w2_ref, b2_ref,
               wc1_ref, bc1_ref, wc2_ref,
               m_ref, t_ref):
    r = r_ref[...]
    d2 = jnp.sum(r * r, axis=1, keepdims=True)
    cat = jnp.concatenate([gs_ref[...], gd_ref[...], d2, a_ref[...]], axis=1)
    m1 = _silu(jnp.dot(cat, w1_ref[...], preferred_element_type=f32)
               + b1_ref[...])
    m = _silu(jnp.dot(m1, w2_ref[...], preferred_element_type=f32)
              + b2_ref[...])
    g = _silu(jnp.dot(m, wc1_ref[...], preferred_element_type=f32)
              + bc1_ref[...])
    wgt = jnp.dot(g, wc2_ref[...], preferred_element_type=f32)[:, :1]
    m_ref[...] = m
    t_ref[...] = r * wgt


def _edge(gs, gd, r, a_ij, w1, b1, w2, b2, wc1, bc1, wc2):
    full = lambda shp: pl.BlockSpec(shp, lambda i: tuple(0 for _ in shp))
    return pl.pallas_call(
        _edge_body,
        grid=(E // BE,),
        in_specs=[
            pl.BlockSpec((BE, DIM), lambda i: (i, 0)),
            pl.BlockSpec((BE, DIM), lambda i: (i, 0)),
            pl.BlockSpec((BE, PW), lambda i: (i, 0)),
            pl.BlockSpec((BE, EDGE_DIM), lambda i: (i, 0)),
            full((S_IN, DIM)),
            full((1, DIM)),
            full((DIM, DIM)),
            full((1, DIM)),
            full((DIM, DIM)),
            full((1, DIM)),
            full((DIM, 8)),
        ],
        out_specs=[
            pl.BlockSpec((BE, DIM), lambda i: (i, 0)),
            pl.BlockSpec((BE, TW), lambda i: (i, 0)),
        ],
        out_shape=[
            jax.ShapeDtypeStruct((E, DIM), f32),
            jax.ShapeDtypeStruct((E, TW), f32),
        ],
    )(gs, gd, r, a_ij, w1, b1, w2, b2, wc1, bc1, wc2)


def _scatter_body(m_hbm, t_hbm, dst_hbm, zm_hbm, zt_hbm,
                  pm_out, pt_out,
                  acc_m, acc_t, di, bm, bt, sem):
    c = lax.axis_index("c")
    s = lax.axis_index("s")
    wid = s * NC + c
    row0 = s * RPT
    pltpu.sync_copy(zm_hbm, acc_m.at[pl.ds(row0, RPT)])
    pltpu.sync_copy(zt_hbm, acc_t.at[pl.ds(row0, RPT)])
    plsc.subcore_barrier()

    nk = (NCHUNKS - wid + NW - 1) // NW

    def body(i, carry):
        k = wid + i * NW
        base = k * CHUNK
        pltpu.sync_copy(dst_hbm.at[pl.ds(base, CHUNK)], di)
        c1 = pltpu.async_copy(m_hbm.at[pl.ds(base, CHUNK)], bm, sem)
        c2 = pltpu.async_copy(t_hbm.at[pl.ds(base, CHUNK)], bt, sem)
        c1.wait()
        c2.wait()
        pltpu.sync_copy(bm, acc_m.at[di], add=True)
        pltpu.sync_copy(bt, acc_t.at[di], add=True)
        return carry

    lax.fori_loop(0, nk, body, 0)
    plsc.subcore_barrier()
    pltpu.sync_copy(acc_m.at[pl.ds(row0, RPT)],
                    pm_out.at[c].at[pl.ds(row0, RPT)])
    pltpu.sync_copy(acc_t.at[pl.ds(row0, RPT)],
                    pt_out.at[c].at[pl.ds(row0, RPT)])


@functools.lru_cache(maxsize=None)
def _scatter_kernel():
    return pl.kernel(
        _scatter_body,
        mesh=_sc_mesh(),
        out_type=[
            jax.ShapeDtypeStruct((NC, NPAD, DIM), f32),
            jax.ShapeDtypeStruct((NC, NPAD, TW), f32),
        ],
        scratch_types=[
            pltpu.VMEM_SHARED((NPAD, DIM), f32),
            pltpu.VMEM_SHARED((NPAD, TW), f32),
            pltpu.VMEM((CHUNK,), i32),
            pltpu.VMEM((CHUNK, DIM), f32),
            pltpu.VMEM((CHUNK, TW), f32),
            pltpu.SemaphoreType.DMA,
        ],
        compiler_params=pltpu.CompilerParams(needs_layout_passes=False,
                                             use_tc_tiling_on_sc=False),
    )


def _scatter(m, t, dst, zm, zt):
    return _scatter_kernel()(m, t, dst, zm, zt)


def _node_body(h_ref, pm0_ref, pm1_ref,
               wn1_ref, bn1_ref, wn2_ref, bn2_ref,
               ho_ref):
    h = h_ref[...]
    agg = pm0_ref[...] + pm1_ref[...]
    cat = jnp.concatenate([h, agg], axis=1)
    u = _silu(jnp.dot(cat, wn1_ref[...], preferred_element_type=f32)
              + bn1_ref[...])
    upd = jnp.dot(u, wn2_ref[...], preferred_element_type=f32) + bn2_ref[...]
    ho_ref[...] = h + upd


def _node(h, pm0, pm1, wn1, bn1, wn2, bn2):
    full = lambda shp: pl.BlockSpec(shp, lambda i: tuple(0 for _ in shp))
    return pl.pallas_call(
        _node_body,
        grid=(N // BN,),
        in_specs=[
            pl.BlockSpec((BN, DIM), lambda i: (i, 0)),
            pl.BlockSpec((BN, DIM), lambda i: (i, 0)),
            pl.BlockSpec((BN, DIM), lambda i: (i, 0)),
            full((2 * DIM, DIM)),
            full((1, DIM)),
            full((DIM, DIM)),
            full((1, DIM)),
        ],
        out_specs=pl.BlockSpec((BN, DIM), lambda i: (i, 0)),
        out_shape=jax.ShapeDtypeStruct((N, DIM), f32),
    )(h, pm0, pm1, wn1, bn1, wn2, bn2)


BNP = 2048


def _pos_body(p_ref, pt_ref, po_ref):
    tsum = pt_ref[0] + pt_ref[1]
    po_ref[...] = p_ref[...] + tsum * (1.0 / AVG_DEG)


def _pos_update(pos, pt):
    return pl.pallas_call(
        _pos_body,
        grid=(NPAD // BNP,),
        in_specs=[
            pl.BlockSpec((BNP, PW), lambda i: (i, 0)),
            pl.BlockSpec((NC, BNP, TW), lambda i: (0, i, 0)),
        ],
        out_specs=pl.BlockSpec((BNP, PW), lambda i: (i, 0)),
        out_shape=jax.ShapeDtypeStruct((NPAD, PW), f32),
    )(pos, pt)


def kernel(h, pos, edge_index, a_ij, We1, be1, We2, be2, Wc1, bc1, Wc2,
           Wn1, bn1, Wn2, bn2):
    src = edge_index[0]
    dst = edge_index[1]
    posp = jnp.pad(pos, ((0, NPAD - N), (0, PW - 3)))
    zm = jnp.zeros((RPT, DIM), f32)
    zt = jnp.zeros((RPT, TW), f32)

    for i in range(NUM_CONVS):
        gs, gd, rel = _gather(h, posp, src, dst)
        m, t = _edge(gs, gd, rel, a_ij,
                     We1[i], be1[i][None],
                     We2[i], be2[i][None], Wc1[i], bc1[i][None],
                     jnp.pad(Wc2[i], ((0, 0), (0, 7))))
        pm, pt = _scatter(m, t, dst, zm, zt)
        posp = _pos_update(posp, pt)
        h = _node(h, pm[0], pm[1], Wn1[i], bn1[i][None],
                  Wn2[i], bn2[i][None])
    return h, posp[:N, :3]


# speedup vs baseline: 3.2981x; 1.1257x over previous
import functools

import jax
import jax.numpy as jnp
from jax import lax
from jax.experimental import pallas as pl
from jax.experimental.pallas import tpu as pltpu
from jax.experimental.pallas import tpu_sc as plsc

N = 10000
E = 320000
DIM = 128
EDGE_DIM = 4
NUM_CONVS = 4
AVG_DEG = E // N

NC = 2
NS = 16
NW = NC * NS
L = 16
CHUNK = 80
NCHUNKS = E // CHUNK
PW = 8
TW = 8
NPAD = 10240
RPT = NPAD // NS

BE = 4000
BN = 2000

f32 = jnp.float32
i32 = jnp.int32


def _silu(x):
    return x * jax.nn.sigmoid(x)


@functools.lru_cache(maxsize=None)
def _sc_mesh():
    return plsc.VectorSubcoreMesh(core_axis_name="c", subcore_axis_name="s",
                                  num_cores=NC, num_subcores=NS)


def _gather_body(xs_hbm, pos_hbm, src_hbm, dst_hbm,
                 gs_out, gd_out, rel_out,
                 pos_v, siA, diA, bsA, bdA, brelA,
                 siB, diB, bsB, bdB, brelB, gsemA, gsemB, wsem):
    c = lax.axis_index("c")
    s = lax.axis_index("s")
    wid = s * NC + c
    nk = (NCHUNKS - wid + NW - 1) // NW

    pltpu.sync_copy(pos_hbm, pos_v)
    lanes = lax.iota(i32, L)
    zero = jnp.zeros((L,), f32)
    for j in range(CHUNK * PW // L):
        r2 = 2 * j + lax.shift_right_logical(lanes, 3)
        c2 = lanes & 7
        plsc.store_scatter(brelA, [r2, c2], zero)
        plsc.store_scatter(brelB, [r2, c2], zero)

    def rel_compute(si, di, brel):
        for g in range(CHUNK // L):
            sv = si[pl.ds(g * L, L)]
            dv = di[pl.ds(g * L, L)]
            rows = g * L + lanes
            for cc in range(3):
                cvec = jnp.full((L,), cc, i32)
                vs = plsc.load_gather(pos_v, [sv, cvec])
                vd = plsc.load_gather(pos_v, [dv, cvec])
                plsc.store_scatter(brel, [rows, cvec], vs - vd)

    def body(j, carry):
        baseA = (wid + 2 * j * NW) * CHUNK
        baseB = baseA + NW * CHUNK
        pltpu.sync_copy(src_hbm.at[pl.ds(baseA, CHUNK)], siA)
        pltpu.sync_copy(dst_hbm.at[pl.ds(baseA, CHUNK)], diA)
        cA1 = pltpu.async_copy(xs_hbm.at[siA], bsA, gsemA)
        cA2 = pltpu.async_copy(xs_hbm.at[diA], bdA, gsemA)
        pltpu.sync_copy(src_hbm.at[pl.ds(baseB, CHUNK)], siB)
        pltpu.sync_copy(dst_hbm.at[pl.ds(baseB, CHUNK)], diB)
        cB1 = pltpu.async_copy(xs_hbm.at[siB], bsB, gsemB)
        cB2 = pltpu.async_copy(xs_hbm.at[diB], bdB, gsemB)
        rel_compute(siA, diA, brelA)
        cA1.wait()
        cA2.wait()
        wA1 = pltpu.async_copy(bsA, gs_out.at[pl.ds(baseA, CHUNK)], wsem)
        wA2 = pltpu.async_copy(bdA, gd_out.at[pl.ds(baseA, CHUNK)], wsem)
        wA3 = pltpu.async_copy(brelA, rel_out.at[pl.ds(baseA, CHUNK)], wsem)
        rel_compute(siB, diB, brelB)
        cB1.wait()
        cB2.wait()
        wB1 = pltpu.async_copy(bsB, gs_out.at[pl.ds(baseB, CHUNK)], wsem)
        wB2 = pltpu.async_copy(bdB, gd_out.at[pl.ds(baseB, CHUNK)], wsem)
        wB3 = pltpu.async_copy(brelB, rel_out.at[pl.ds(baseB, CHUNK)], wsem)
        wA1.wait(); wA2.wait(); wA3.wait()
        wB1.wait(); wB2.wait(); wB3.wait()
        return carry

    lax.fori_loop(0, nk // 2, body, 0)

    @pl.when(nk % 2 == 1)
    def _tail():
        base = (wid + (nk - 1) * NW) * CHUNK
        pltpu.sync_copy(src_hbm.at[pl.ds(base, CHUNK)], siA)
        pltpu.sync_copy(dst_hbm.at[pl.ds(base, CHUNK)], diA)
        c1 = pltpu.async_copy(xs_hbm.at[siA], bsA, gsemA)
        c2 = pltpu.async_copy(xs_hbm.at[diA], bdA, gsemA)
        rel_compute(siA, diA, brelA)
        c1.wait()
        c2.wait()
        w1 = pltpu.async_copy(bsA, gs_out.at[pl.ds(base, CHUNK)], wsem)
        w2 = pltpu.async_copy(bdA, gd_out.at[pl.ds(base, CHUNK)], wsem)
        w3 = pltpu.async_copy(brelA, rel_out.at[pl.ds(base, CHUNK)], wsem)
        w1.wait(); w2.wait(); w3.wait()


@functools.lru_cache(maxsize=None)
def _gather_kernel():
    return pl.kernel(
        _gather_body,
        mesh=_sc_mesh(),
        out_type=[
            jax.ShapeDtypeStruct((E, DIM), f32),
            jax.ShapeDtypeStruct((E, DIM), f32),
            jax.ShapeDtypeStruct((E, PW), f32),
        ],
        scratch_types=[
            pltpu.VMEM((NPAD, 4), f32),
            pltpu.VMEM((CHUNK,), i32),
            pltpu.VMEM((CHUNK,), i32),
            pltpu.VMEM((CHUNK, DIM), f32),
            pltpu.VMEM((CHUNK, DIM), f32),
            pltpu.VMEM((CHUNK, PW), f32),
            pltpu.VMEM((CHUNK,), i32),
            pltpu.VMEM((CHUNK,), i32),
            pltpu.VMEM((CHUNK, DIM), f32),
            pltpu.VMEM((CHUNK, DIM), f32),
            pltpu.VMEM((CHUNK, PW), f32),
            pltpu.SemaphoreType.DMA,
            pltpu.SemaphoreType.DMA,
            pltpu.SemaphoreType.DMA,
        ],
        compiler_params=pltpu.CompilerParams(needs_layout_passes=False,
                                             use_tc_tiling_on_sc=False),
    )


def _gather(h, pos4, src, dst):
    return _gather_kernel()(h, pos4, src, dst)


S_IN = 2 * DIM + 1 + EDGE_DIM


def _edge_body(gs_ref, gd_ref, r_ref, a_ref,
               w1_ref, b1_ref, w2_ref, b2_ref,
               wc1_ref, bc1_ref, wc2_ref,
               m_ref, t_ref):
    r = r_ref[...]
    d2 = jnp.sum(r * r, axis=1, keepdims=True)
    cat = jnp.concatenate([gs_ref[...], gd_ref[...], d2, a_ref[...]], axis=1)
    m1 = _silu(jnp.dot(cat, w1_ref[...], preferred_element_type=f32)
               + b1_ref[...])
    m = _silu(jnp.dot(m1, w2_ref[...], preferred_element_type=f32)
              + b2_ref[...])
    g = _silu(jnp.dot(m, wc1_ref[...], preferred_element_type=f32)
              + bc1_ref[...])
    wgt = jnp.dot(g, wc2_ref[...], preferred_element_type=f32)[:, :1]
    m_ref[...] = m
    t_ref[...] = r * wgt


def _edge(gs, gd, r, a_ij, w1, b1, w2, b2, wc1, bc1, wc2):
    full = lambda shp: pl.BlockSpec(shp, lambda i: tuple(0 for _ in shp))
    return pl.pallas_call(
        _edge_body,
        grid=(E // BE,),
        in_specs=[
            pl.BlockSpec((BE, DIM), lambda i: (i, 0)),
            pl.BlockSpec((BE, DIM), lambda i: (i, 0)),
            pl.BlockSpec((BE, PW), lambda i: (i, 0)),
            pl.BlockSpec((BE, EDGE_DIM), lambda i: (i, 0)),
            full((S_IN, DIM)),
            full((1, DIM)),
            full((DIM, DIM)),
            full((1, DIM)),
            full((DIM, DIM)),
            full((1, DIM)),
            full((DIM, 8)),
        ],
        out_specs=[
            pl.BlockSpec((BE, DIM), lambda i: (i, 0)),
            pl.BlockSpec((BE, TW), lambda i: (i, 0)),
        ],
        out_shape=[
            jax.ShapeDtypeStruct((E, DIM), f32),
            jax.ShapeDtypeStruct((E, TW), f32),
        ],
    )(gs, gd, r, a_ij, w1, b1, w2, b2, wc1, bc1, wc2)


def _scatter_body(m_hbm, t_hbm, dst_hbm, zm_hbm, zt_hbm,
                  pm_out, pt_out,
                  acc_m, acc_t, di, bm, bt, sem):
    c = lax.axis_index("c")
    s = lax.axis_index("s")
    wid = s * NC + c
    row0 = s * RPT
    pltpu.sync_copy(zm_hbm, acc_m.at[pl.ds(row0, RPT)])
    pltpu.sync_copy(zt_hbm, acc_t.at[pl.ds(row0, RPT)])
    plsc.subcore_barrier()

    nk = (NCHUNKS - wid + NW - 1) // NW

    def body(i, carry):
        k = wid + i * NW
        base = k * CHUNK
        pltpu.sync_copy(dst_hbm.at[pl.ds(base, CHUNK)], di)
        c1 = pltpu.async_copy(m_hbm.at[pl.ds(base, CHUNK)], bm, sem)
        c2 = pltpu.async_copy(t_hbm.at[pl.ds(base, CHUNK)], bt, sem)
        c1.wait()
        c2.wait()
        pltpu.sync_copy(bm, acc_m.at[di], add=True)
        pltpu.sync_copy(bt, acc_t.at[di], add=True)
        return carry

    lax.fori_loop(0, nk, body, 0)
    plsc.subcore_barrier()
    pltpu.sync_copy(acc_m.at[pl.ds(row0, RPT)],
                    pm_out.at[c].at[pl.ds(row0, RPT)])
    pltpu.sync_copy(acc_t.at[pl.ds(row0, RPT)],
                    pt_out.at[c].at[pl.ds(row0, RPT)])


@functools.lru_cache(maxsize=None)
def _scatter_kernel():
    return pl.kernel(
        _scatter_body,
        mesh=_sc_mesh(),
        out_type=[
            jax.ShapeDtypeStruct((NC, NPAD, DIM), f32),
            jax.ShapeDtypeStruct((NC, NPAD, TW), f32),
        ],
        scratch_types=[
            pltpu.VMEM_SHARED((NPAD, DIM), f32),
            pltpu.VMEM_SHARED((NPAD, TW), f32),
            pltpu.VMEM((CHUNK,), i32),
            pltpu.VMEM((CHUNK, DIM), f32),
            pltpu.VMEM((CHUNK, TW), f32),
            pltpu.SemaphoreType.DMA,
        ],
        compiler_params=pltpu.CompilerParams(needs_layout_passes=False,
                                             use_tc_tiling_on_sc=False),
    )


def _scatter(m, t, dst, zm, zt):
    return _scatter_kernel()(m, t, dst, zm, zt)


def _node_body(h_ref, pm0_ref, pm1_ref,
               wn1_ref, bn1_ref, wn2_ref, bn2_ref,
               ho_ref):
    h = h_ref[...]
    agg = pm0_ref[...] + pm1_ref[...]
    cat = jnp.concatenate([h, agg], axis=1)
    u = _silu(jnp.dot(cat, wn1_ref[...], preferred_element_type=f32)
              + bn1_ref[...])
    upd = jnp.dot(u, wn2_ref[...], preferred_element_type=f32) + bn2_ref[...]
    ho_ref[...] = h + upd


def _node(h, pm0, pm1, wn1, bn1, wn2, bn2):
    full = lambda shp: pl.BlockSpec(shp, lambda i: tuple(0 for _ in shp))
    return pl.pallas_call(
        _node_body,
        grid=(N // BN,),
        in_specs=[
            pl.BlockSpec((BN, DIM), lambda i: (i, 0)),
            pl.BlockSpec((BN, DIM), lambda i: (i, 0)),
            pl.BlockSpec((BN, DIM), lambda i: (i, 0)),
            full((2 * DIM, DIM)),
            full((1, DIM)),
            full((DIM, DIM)),
            full((1, DIM)),
        ],
        out_specs=pl.BlockSpec((BN, DIM), lambda i: (i, 0)),
        out_shape=jax.ShapeDtypeStruct((N, DIM), f32),
    )(h, pm0, pm1, wn1, bn1, wn2, bn2)


BNP = 2048


def _pos_body(p_ref, pt_ref, po_ref):
    tsum = pt_ref[0] + pt_ref[1]
    po_ref[...] = p_ref[...] + tsum[:, :4] * (1.0 / AVG_DEG)


def _pos_update(pos, pt):
    return pl.pallas_call(
        _pos_body,
        grid=(NPAD // BNP,),
        in_specs=[
            pl.BlockSpec((BNP, 4), lambda i: (i, 0)),
            pl.BlockSpec((NC, BNP, TW), lambda i: (0, i, 0)),
        ],
        out_specs=pl.BlockSpec((BNP, 4), lambda i: (i, 0)),
        out_shape=jax.ShapeDtypeStruct((NPAD, 4), f32),
    )(pos, pt)


def kernel(h, pos, edge_index, a_ij, We1, be1, We2, be2, Wc1, bc1, Wc2,
           Wn1, bn1, Wn2, bn2):
    src = edge_index[0]
    dst = edge_index[1]
    posp = jnp.pad(pos, ((0, NPAD - N), (0, 1)))
    zm = jnp.zeros((RPT, DIM), f32)
    zt = jnp.zeros((RPT, TW), f32)

    for i in range(NUM_CONVS):
        gs, gd, rel = _gather(h, posp, src, dst)
        m, t = _edge(gs, gd, rel, a_ij,
                     We1[i], be1[i][None],
                     We2[i], be2[i][None], Wc1[i], bc1[i][None],
                     jnp.pad(Wc2[i], ((0, 0), (0, 7))))
        pm, pt = _scatter(m, t, dst, zm, zt)
        posp = _pos_update(posp, pt)
        h = _node(h, pm[0], pm[1], Wn1[i], bn1[i][None],
                  Wn2[i], bn2[i][None])
    return h, posp[:N, :3]


# speedup vs baseline: 3.5452x; 1.0749x over previous
import functools

import jax
import jax.numpy as jnp
from jax import lax
from jax.experimental import pallas as pl
from jax.experimental.pallas import tpu as pltpu
from jax.experimental.pallas import tpu_sc as plsc

N = 10000
E = 320000
DIM = 128
EDGE_DIM = 4
NUM_CONVS = 4
AVG_DEG = E // N

NC = 2
NS = 16
NW = NC * NS
L = 16
CHUNK = 80
NCHUNKS = E // CHUNK
PW = 8
TW = 8
NPAD = 10240
RPT = NPAD // NS

BE = 4000
BN = 2000

f32 = jnp.float32
i32 = jnp.int32


def _silu(x):
    return x * jax.nn.sigmoid(x)


@functools.lru_cache(maxsize=None)
def _sc_mesh():
    return plsc.VectorSubcoreMesh(core_axis_name="c", subcore_axis_name="s",
                                  num_cores=NC, num_subcores=NS)


def _gather_body(xs_hbm, pos_hbm, src_hbm, dst_hbm,
                 gs_out, gd_out, rel_out,
                 pos_v, siA, diA, bsA, bdA, brelA,
                 siB, diB, bsB, bdB, brelB, gsemA, gsemB, wsem):
    c = lax.axis_index("c")
    s = lax.axis_index("s")
    wid = s * NC + c
    nk = (NCHUNKS - wid + NW - 1) // NW

    pltpu.sync_copy(pos_hbm, pos_v)
    lanes = lax.iota(i32, L)
    zero = jnp.zeros((L,), f32)
    for j in range(CHUNK * PW // L):
        r2 = 2 * j + lax.shift_right_logical(lanes, 3)
        c2 = lanes & 7
        plsc.store_scatter(brelA, [r2, c2], zero)
        plsc.store_scatter(brelB, [r2, c2], zero)

    def rel_compute(si, di, brel):
        for g in range(CHUNK // L):
            sv = si[pl.ds(g * L, L)]
            dv = di[pl.ds(g * L, L)]
            rows = g * L + lanes
            for cc in range(3):
                cvec = jnp.full((L,), cc, i32)
                vs = plsc.load_gather(pos_v, [sv, cvec])
                vd = plsc.load_gather(pos_v, [dv, cvec])
                plsc.store_scatter(brel, [rows, cvec], vs - vd)

    def body(j, carry):
        baseA = (wid + 2 * j * NW) * CHUNK
        baseB = baseA + NW * CHUNK
        pltpu.sync_copy(src_hbm.at[pl.ds(baseA, CHUNK)], siA)
        pltpu.sync_copy(dst_hbm.at[pl.ds(baseA, CHUNK)], diA)
        cA1 = pltpu.async_copy(xs_hbm.at[siA], bsA, gsemA)
        cA2 = pltpu.async_copy(xs_hbm.at[diA], bdA, gsemA)
        pltpu.sync_copy(src_hbm.at[pl.ds(baseB, CHUNK)], siB)
        pltpu.sync_copy(dst_hbm.at[pl.ds(baseB, CHUNK)], diB)
        cB1 = pltpu.async_copy(xs_hbm.at[siB], bsB, gsemB)
        cB2 = pltpu.async_copy(xs_hbm.at[diB], bdB, gsemB)
        rel_compute(siA, diA, brelA)
        cA1.wait()
        cA2.wait()
        wA1 = pltpu.async_copy(bsA, gs_out.at[pl.ds(baseA, CHUNK)], wsem)
        wA2 = pltpu.async_copy(bdA, gd_out.at[pl.ds(baseA, CHUNK)], wsem)
        wA3 = pltpu.async_copy(brelA, rel_out.at[pl.ds(baseA, CHUNK)], wsem)
        rel_compute(siB, diB, brelB)
        cB1.wait()
        cB2.wait()
        wB1 = pltpu.async_copy(bsB, gs_out.at[pl.ds(baseB, CHUNK)], wsem)
        wB2 = pltpu.async_copy(bdB, gd_out.at[pl.ds(baseB, CHUNK)], wsem)
        wB3 = pltpu.async_copy(brelB, rel_out.at[pl.ds(baseB, CHUNK)], wsem)
        wA1.wait(); wA2.wait(); wA3.wait()
        wB1.wait(); wB2.wait(); wB3.wait()
        return carry

    lax.fori_loop(0, nk // 2, body, 0)

    @pl.when(nk % 2 == 1)
    def _tail():
        base = (wid + (nk - 1) * NW) * CHUNK
        pltpu.sync_copy(src_hbm.at[pl.ds(base, CHUNK)], siA)
        pltpu.sync_copy(dst_hbm.at[pl.ds(base, CHUNK)], diA)
        c1 = pltpu.async_copy(xs_hbm.at[siA], bsA, gsemA)
        c2 = pltpu.async_copy(xs_hbm.at[diA], bdA, gsemA)
        rel_compute(siA, diA, brelA)
        c1.wait()
        c2.wait()
        w1 = pltpu.async_copy(bsA, gs_out.at[pl.ds(base, CHUNK)], wsem)
        w2 = pltpu.async_copy(bdA, gd_out.at[pl.ds(base, CHUNK)], wsem)
        w3 = pltpu.async_copy(brelA, rel_out.at[pl.ds(base, CHUNK)], wsem)
        w1.wait(); w2.wait(); w3.wait()


@functools.lru_cache(maxsize=None)
def _gather_kernel():
    return pl.kernel(
        _gather_body,
        mesh=_sc_mesh(),
        out_type=[
            jax.ShapeDtypeStruct((E, DIM), f32),
            jax.ShapeDtypeStruct((E, DIM), f32),
            jax.ShapeDtypeStruct((E, PW), f32),
        ],
        scratch_types=[
            pltpu.VMEM((NPAD, 4), f32),
            pltpu.VMEM((CHUNK,), i32),
            pltpu.VMEM((CHUNK,), i32),
            pltpu.VMEM((CHUNK, DIM), f32),
            pltpu.VMEM((CHUNK, DIM), f32),
            pltpu.VMEM((CHUNK, PW), f32),
            pltpu.VMEM((CHUNK,), i32),
            pltpu.VMEM((CHUNK,), i32),
            pltpu.VMEM((CHUNK, DIM), f32),
            pltpu.VMEM((CHUNK, DIM), f32),
            pltpu.VMEM((CHUNK, PW), f32),
            pltpu.SemaphoreType.DMA,
            pltpu.SemaphoreType.DMA,
            pltpu.SemaphoreType.DMA,
        ],
        compiler_params=pltpu.CompilerParams(needs_layout_passes=False,
                                             use_tc_tiling_on_sc=False),
    )


def _gather(h, pos4, src, dst):
    return _gather_kernel()(h, pos4, src, dst)


S_IN = 2 * DIM + 1 + EDGE_DIM


def _edge_body(gs_ref, gd_ref, r_ref, a_ref,
               w1_ref, b1_ref, w2_ref, b2_ref,
               wc1_ref, bc1_ref, wc2_ref,
               m_ref, t_ref):
    r = r_ref[...]
    d2 = jnp.sum(r * r, axis=1, keepdims=True)
    cat = jnp.concatenate([gs_ref[...], gd_ref[...], d2, a_ref[...]], axis=1)
    m1 = _silu(jnp.dot(cat, w1_ref[...], preferred_element_type=f32)
               + b1_ref[...])
    m = _silu(jnp.dot(m1, w2_ref[...], preferred_element_type=f32)
              + b2_ref[...])
    g = _silu(jnp.dot(m, wc1_ref[...], preferred_element_type=f32)
              + bc1_ref[...])
    wgt = jnp.dot(g, wc2_ref[...], preferred_element_type=f32)[:, :1]
    m_ref[...] = m
    t_ref[...] = r * wgt


def _edge(gs, gd, r, a_ij, w1, b1, w2, b2, wc1, bc1, wc2):
    full = lambda shp: pl.BlockSpec(shp, lambda i: tuple(0 for _ in shp))
    return pl.pallas_call(
        _edge_body,
        grid=(E // BE,),
        in_specs=[
            pl.BlockSpec((BE, DIM), lambda i: (i, 0)),
            pl.BlockSpec((BE, DIM), lambda i: (i, 0)),
            pl.BlockSpec((BE, PW), lambda i: (i, 0)),
            pl.BlockSpec((BE, EDGE_DIM), lambda i: (i, 0)),
            full((S_IN, DIM)),
            full((1, DIM)),
            full((DIM, DIM)),
            full((1, DIM)),
            full((DIM, DIM)),
            full((1, DIM)),
            full((DIM, 8)),
        ],
        out_specs=[
            pl.BlockSpec((BE, DIM), lambda i: (i, 0)),
            pl.BlockSpec((BE, TW), lambda i: (i, 0)),
        ],
        out_shape=[
            jax.ShapeDtypeStruct((E, DIM), f32),
            jax.ShapeDtypeStruct((E, TW), f32),
        ],
    )(gs, gd, r, a_ij, w1, b1, w2, b2, wc1, bc1, wc2)


def _scatter_body(m_hbm, t_hbm, dst_hbm, zm_hbm, zt_hbm,
                  pm_out, pt_out,
                  acc_m, acc_t, diA, bmA, btA, diB, bmB, btB, semA, semB):
    c = lax.axis_index("c")
    s = lax.axis_index("s")
    wid = s * NC + c
    row0 = s * RPT
    pltpu.sync_copy(zm_hbm, acc_m.at[pl.ds(row0, RPT)])
    pltpu.sync_copy(zt_hbm, acc_t.at[pl.ds(row0, RPT)])
    plsc.subcore_barrier()

    nk = (NCHUNKS - wid + NW - 1) // NW

    def body(j, carry):
        baseA = (wid + 2 * j * NW) * CHUNK
        baseB = baseA + NW * CHUNK
        pltpu.sync_copy(dst_hbm.at[pl.ds(baseA, CHUNK)], diA)
        cA1 = pltpu.async_copy(m_hbm.at[pl.ds(baseA, CHUNK)], bmA, semA)
        cA2 = pltpu.async_copy(t_hbm.at[pl.ds(baseA, CHUNK)], btA, semA)
        pltpu.sync_copy(dst_hbm.at[pl.ds(baseB, CHUNK)], diB)
        cB1 = pltpu.async_copy(m_hbm.at[pl.ds(baseB, CHUNK)], bmB, semB)
        cB2 = pltpu.async_copy(t_hbm.at[pl.ds(baseB, CHUNK)], btB, semB)
        cA1.wait()
        cA2.wait()
        pltpu.sync_copy(bmA, acc_m.at[diA], add=True)
        pltpu.sync_copy(btA, acc_t.at[diA], add=True)
        cB1.wait()
        cB2.wait()
        pltpu.sync_copy(bmB, acc_m.at[diB], add=True)
        pltpu.sync_copy(btB, acc_t.at[diB], add=True)
        return carry

    lax.fori_loop(0, nk // 2, body, 0)

    @pl.when(nk % 2 == 1)
    def _tail():
        base = (wid + (nk - 1) * NW) * CHUNK
        pltpu.sync_copy(dst_hbm.at[pl.ds(base, CHUNK)], diA)
        c1 = pltpu.async_copy(m_hbm.at[pl.ds(base, CHUNK)], bmA, semA)
        c2 = pltpu.async_copy(t_hbm.at[pl.ds(base, CHUNK)], btA, semA)
        c1.wait()
        c2.wait()
        pltpu.sync_copy(bmA, acc_m.at[diA], add=True)
        pltpu.sync_copy(btA, acc_t.at[diA], add=True)

    plsc.subcore_barrier()
    pltpu.sync_copy(acc_m.at[pl.ds(row0, RPT)],
                    pm_out.at[c].at[pl.ds(row0, RPT)])
    pltpu.sync_copy(acc_t.at[pl.ds(row0, RPT)],
                    pt_out.at[c].at[pl.ds(row0, RPT)])


@functools.lru_cache(maxsize=None)
def _scatter_kernel():
    return pl.kernel(
        _scatter_body,
        mesh=_sc_mesh(),
        out_type=[
            jax.ShapeDtypeStruct((NC, NPAD, DIM), f32),
            jax.ShapeDtypeStruct((NC, NPAD, TW), f32),
        ],
        scratch_types=[
            pltpu.VMEM_SHARED((NPAD, DIM), f32),
            pltpu.VMEM_SHARED((NPAD, TW), f32),
            pltpu.VMEM((CHUNK,), i32),
            pltpu.VMEM((CHUNK, DIM), f32),
            pltpu.VMEM((CHUNK, TW), f32),
            pltpu.VMEM((CHUNK,), i32),
            pltpu.VMEM((CHUNK, DIM), f32),
            pltpu.VMEM((CHUNK, TW), f32),
            pltpu.SemaphoreType.DMA,
            pltpu.SemaphoreType.DMA,
        ],
        compiler_params=pltpu.CompilerParams(needs_layout_passes=False,
                                             use_tc_tiling_on_sc=False),
    )


def _scatter(m, t, dst, zm, zt):
    return _scatter_kernel()(m, t, dst, zm, zt)


def _node_body(h_ref, pm0_ref, pm1_ref,
               wn1_ref, bn1_ref, wn2_ref, bn2_ref,
               ho_ref):
    h = h_ref[...]
    agg = pm0_ref[...] + pm1_ref[...]
    cat = jnp.concatenate([h, agg], axis=1)
    u = _silu(jnp.dot(cat, wn1_ref[...], preferred_element_type=f32)
              + bn1_ref[...])
    upd = jnp.dot(u, wn2_ref[...], preferred_element_type=f32) + bn2_ref[...]
    ho_ref[...] = h + upd


def _node(h, pm0, pm1, wn1, bn1, wn2, bn2):
    full = lambda shp: pl.BlockSpec(shp, lambda i: tuple(0 for _ in shp))
    return pl.pallas_call(
        _node_body,
        grid=(N // BN,),
        in_specs=[
            pl.BlockSpec((BN, DIM), lambda i: (i, 0)),
            pl.BlockSpec((BN, DIM), lambda i: (i, 0)),
            pl.BlockSpec((BN, DIM), lambda i: (i, 0)),
            full((2 * DIM, DIM)),
            full((1, DIM)),
            full((DIM, DIM)),
            full((1, DIM)),
        ],
        out_specs=pl.BlockSpec((BN, DIM), lambda i: (i, 0)),
        out_shape=jax.ShapeDtypeStruct((N, DIM), f32),
    )(h, pm0, pm1, wn1, bn1, wn2, bn2)


BNP = 2048


def _pos_body(p_ref, pt_ref, po_ref):
    tsum = pt_ref[0] + pt_ref[1]
    po_ref[...] = p_ref[...] + tsum[:, :4] * (1.0 / AVG_DEG)


def _pos_update(pos, pt):
    return pl.pallas_call(
        _pos_body,
        grid=(NPAD // BNP,),
        in_specs=[
            pl.BlockSpec((BNP, 4), lambda i: (i, 0)),
            pl.BlockSpec((NC, BNP, TW), lambda i: (0, i, 0)),
        ],
        out_specs=pl.BlockSpec((BNP, 4), lambda i: (i, 0)),
        out_shape=jax.ShapeDtypeStruct((NPAD, 4), f32),
    )(pos, pt)


def kernel(h, pos, edge_index, a_ij, We1, be1, We2, be2, Wc1, bc1, Wc2,
           Wn1, bn1, Wn2, bn2):
    src = edge_index[0]
    dst = edge_index[1]
    posp = jnp.pad(pos, ((0, NPAD - N), (0, 1)))
    zm = jnp.zeros((RPT, DIM), f32)
    zt = jnp.zeros((RPT, TW), f32)

    for i in range(NUM_CONVS):
        gs, gd, rel = _gather(h, posp, src, dst)
        m, t = _edge(gs, gd, rel, a_ij,
                     We1[i], be1[i][None],
                     We2[i], be2[i][None], Wc1[i], bc1[i][None],
                     jnp.pad(Wc2[i], ((0, 0), (0, 7))))
        pm, pt = _scatter(m, t, dst, zm, zt)
        posp = _pos_update(posp, pt)
        h = _node(h, pm[0], pm[1], Wn1[i], bn1[i][None],
                  Wn2[i], bn2[i][None])
    return h, posp[:N, :3]
